# Initial kernel scaffold; baseline (speedup 1.0000x reference)
import jax
import jax.numpy as jnp
from jax.experimental import pallas as pl


def kernel(x, idx_token, agg_weight, loc_orig, skip_w, conv_w, conv_b, ms_skip_w, dw_w, dw_b, pw_w, pw_b, norm_g, norm_b, norm1_g, norm1_b, score_w, score_b):
    raise NotImplementedError("write your pallas kernel here")


# speedup vs baseline: 1.9490x; 1.9490x over previous
import jax
import jax.numpy as jnp
from jax.experimental import pallas as pl
from jax.experimental.pallas import tpu as pltpu

_B, _H, _W = 16, 32, 32
_N = _H * _W
_CIN, _CO = 192, 384
_CL, _K = 256, 5
_DEF = jax.lax.Precision.DEFAULT
_HI = jax.lax.Precision.HIGHEST
_DN = (((1,), (0,)), ((), ()))
_DNT = (((1,), (1,)), ((), ()))


def _bs(shape):
    nd = len(shape)
    return pl.BlockSpec((1,) + shape, lambda b: (b,) + (0,) * nd)


def _bw(shape):
    nd = len(shape)
    return pl.BlockSpec(shape, lambda b: (0,) * nd)


def _proj_kernel(x_ref, skw_ref, msw_ref, xs_ref, xm_ref):
    x = x_ref[0]
    xs_ref[0] = jax.lax.dot_general(x, skw_ref[...], _DN, precision=_DEF,
                                    preferred_element_type=jnp.float32)
    xm_ref[0] = jax.lax.dot_general(x, msw_ref[...], _DN, precision=_DEF,
                                    preferred_element_type=jnp.float32)


def _norm_kernel(tok_ref, m_ref, v_ref, g_ref, b_ref, sw_ref, sb_ref,
                 xn_ref, ts_ref):
    xn = (tok_ref[0] - m_ref[0]) / jnp.sqrt(v_ref[0] + 1e-5) * g_ref[0] + b_ref[0]
    xn_ref[0] = xn
    ts_ref[0] = jnp.sum(xn * sw_ref[0], axis=-1, keepdims=True) + sb_ref[0, 0]


def _knn_kernel(xn_ref, x2c_ref, x2r_ref, dist_ref, d5_ref, dmax_ref):
    def body(i, mx):
        xc = xn_ref[0, pl.ds(i * 128, 128), :]
        e = jax.lax.dot_general(xc, xn_ref[0], _DNT, precision=_DEF,
                                preferred_element_type=jnp.float32)
        d2 = (x2c_ref[0, pl.ds(i * 128, 128), :] + x2r_ref[0]) - 2.0 * e
        dist = jnp.sqrt(jnp.maximum(d2, 0.0)) / (384.0 ** 0.5)
        dist_ref[0, pl.ds(i * 128, 128), :] = dist
        col = jax.lax.broadcasted_iota(jnp.int32, (128, _N), 1)
        cur = dist
        vals = []
        for t in range(_K):
            mv = jnp.min(cur, axis=-1, keepdims=True)
            vals.append(mv)
            if t < _K - 1:
                idx = jnp.min(jnp.where(cur == mv, col, _N), axis=-1, keepdims=True)
                cur = jnp.where(col == idx, jnp.float32(jnp.inf), cur)
        d5_ref[0, pl.ds(i * 128, 128), :] = jnp.concatenate(vals, axis=1)
        return jnp.maximum(mx, jnp.max(dist, axis=(0, 1), keepdims=True))

    mx = jax.lax.fori_loop(0, _N // 128, body,
                           jnp.full((1, 1), -jnp.inf, jnp.float32))
    dmax_ref[0] = mx


def _score_kernel(dist_ref, dc_ref, dr_ref, dmax_ref, sc_ref):
    ch = dist_ref[0]
    dci = dc_ref[0]
    dmin = jnp.min(jnp.where(dr_ref[0] > dci, ch, dmax_ref[0]),
                   axis=-1, keepdims=True)
    sc_ref[0] = dmin * dci


def _rank_kernel(sc_ref, srow_ref, rk_ref):
    j = pl.program_id(1)
    sci = sc_ref[0]
    s_row = srow_ref[0]
    lane = jax.lax.broadcasted_iota(jnp.int32, (128, _N), 1)
    sub = jax.lax.broadcasted_iota(jnp.int32, (128, _N), 0) + j * 128
    gt = (s_row > sci) | ((s_row == sci) & (lane < sub))
    rk_ref[0] = jnp.sum(gt.astype(jnp.float32), axis=-1, keepdims=True)


def _assign_kernel(dist_ref, rk_ref, rrow_ref, idx_ref):
    ch = dist_ref[0]
    rank_row = rrow_ref[0]
    masked = jnp.where(rank_row < jnp.float32(_CL), ch, jnp.float32(jnp.inf))
    mmin = jnp.min(masked, axis=-1, keepdims=True)
    cand = jnp.where(masked == mmin, rank_row, jnp.float32(_N))
    asg = jnp.min(cand, axis=-1, keepdims=True)
    rci = rk_ref[0]
    idx_ref[0] = jnp.where(rci < jnp.float32(_CL), rci, asg)


def _merge_kernel(xn_ref, idx_ref, ts_ref, out_ref):
    def body4(i, aw):
        idxc = idx_ref[0, pl.ds(i * 128, 128), :].astype(jnp.int32)
        twc = jnp.exp(ts_ref[0, pl.ds(i * 128, 128), :])
        ohc = (jax.lax.broadcasted_iota(jnp.int32, (128, _CL), 1) == idxc
               ).astype(jnp.float32)
        return aw + jnp.sum(ohc * twc, axis=0, keepdims=True)

    all_w = jax.lax.fori_loop(0, _N // 128, body4,
                              jnp.zeros((1, _CL), jnp.float32)) + 1e-6

    def body5(i, acc):
        idxc = idx_ref[0, pl.ds(i * 128, 128), :].astype(jnp.int32)
        twc = jnp.exp(ts_ref[0, pl.ds(i * 128, 128), :])
        ohc = (jax.lax.broadcasted_iota(jnp.int32, (128, _CL), 1) == idxc
               ).astype(jnp.float32)
        gath = jnp.sum(ohc * all_w, axis=1, keepdims=True)
        xw = xn_ref[0, pl.ds(i * 128, 128), :] * (twc / gath)
        return acc + jax.lax.dot_general(ohc, xw, (((0,), (0,)), ((), ())),
                                         precision=_HI,
                                         preferred_element_type=jnp.float32)

    out_ref[0] = jax.lax.fori_loop(0, _N // 128, body5,
                                   jnp.zeros((_CL, _CO), jnp.float32))


def _ms_kernel(p00_ref, p01_ref, p10_ref, p11_ref, dwr_ref, dwb_ref, pwt_ref,
               pwb_ref, x3_ref, g_ref, b_ref, out_ref):
    planes = ((p00_ref, p01_ref), (p10_ref, p11_ref))
    acc = jnp.zeros((16, 16, _CO), jnp.float32)
    for ty in range(5):
        for tx in range(5):
            plane = planes[ty % 2][tx % 2][0]
            r0, c0 = ty // 2, tx // 2
            sl = jax.lax.slice(plane, (r0, c0, 0), (r0 + 16, c0 + 16, _CO))
            acc = acc + sl * dwr_ref[5 * ty + tx][None, None, :]
    acc = acc + dwb_ref[0][None, :]
    accf = acc.reshape(256, _CO)
    y = jax.lax.dot_general(accf, pwt_ref[...], _DN, precision=_HI,
                            preferred_element_type=jnp.float32)
    y = y + pwb_ref[0] + x3_ref[0]
    m = jnp.mean(y, axis=-1, keepdims=True)
    v = jnp.mean((y - m) ** 2, axis=-1, keepdims=True)
    out_ref[0] = (y - m) / jnp.sqrt(v + 1e-5) * g_ref[0] + b_ref[0]


def kernel(x, idx_token, agg_weight, loc_orig, skip_w, conv_w, conv_b,
           ms_skip_w, dw_w, dw_b, pw_w, pw_b, norm_g, norm_b, norm1_g,
           norm1_b, score_w, score_b):
    f32 = jnp.float32
    B, N, C_in, C_out = _B, _N, _CIN, _CO

    xs, xm = pl.pallas_call(
        _proj_kernel,
        grid=(B,),
        in_specs=[_bs((N, C_in)), _bw((C_in, C_out)), _bw((C_in, C_out))],
        out_specs=[_bs((N, C_out)), _bs((N, C_out))],
        out_shape=[jax.ShapeDtypeStruct((B, N, C_out), f32),
                   jax.ShapeDtypeStruct((B, N, C_out), f32)],
    )(x, skip_w.T, ms_skip_w.T)

    x_map0 = x.reshape(B, _H, _W, C_in).transpose(0, 3, 1, 2)
    x_map = jax.lax.conv_general_dilated(
        x_map0, conv_w, (2, 2), [(1, 1), (1, 1)],
        dimension_numbers=('NCHW', 'OIHW', 'NCHW'), feature_group_count=1)
    x_map = x_map + conv_b[None, :, None, None]

    Hm, Wm = 16, 16
    loc = jnp.clip(loc_orig, -1.0, 1.0)
    loc = 0.5 * (loc + 1.0) * jnp.array([Wm, Hm], f32)[None, None, :] - 0.5
    loc = jnp.round(loc).astype(jnp.int32)
    xi = jnp.clip(loc[..., 0], 0, Wm - 1)
    yi = jnp.clip(loc[..., 1], 0, Hm - 1)
    idx_hw = yi * Wm + xi
    fm = x_map.transpose(0, 2, 3, 1).reshape(B, Hm * Wm, C_out)
    vals = jnp.take_along_axis(fm, idx_hw[..., None], axis=1)
    ones = jnp.ones(idx_hw.shape, f32)

    def _per_batch(v, idx, w):
        cnt = jax.ops.segment_sum(w, idx, num_segments=N) + 1e-6
        s = jax.ops.segment_sum(v, idx, num_segments=N)
        return s / cnt[:, None]

    x_tok = xs + jax.vmap(_per_batch)(vals, idx_token, ones)

    m = jnp.mean(x_tok, -1, keepdims=True)
    v = jnp.mean((x_tok - m) ** 2, -1, keepdims=True)

    xn, ts = pl.pallas_call(
        _norm_kernel,
        grid=(B,),
        in_specs=[_bs((N, C_out)), _bs((N, 1)), _bs((N, 1)), _bw((1, C_out)),
                  _bw((1, C_out)), _bw((1, C_out)), _bw((1, 1))],
        out_specs=[_bs((N, C_out)), _bs((N, 1))],
        out_shape=[jax.ShapeDtypeStruct((B, N, C_out), f32),
                   jax.ShapeDtypeStruct((B, N, 1), f32)],
    )(x_tok, m, v, norm_g.reshape(1, C_out), norm_b.reshape(1, C_out),
      score_w.reshape(1, C_out), score_b.reshape(1, 1))

    x2 = jnp.sum(xn * xn, -1)
    x2c = x2.reshape(B, N, 1)
    x2r = x2.reshape(B, 1, N)

    dist, d5, dmax = pl.pallas_call(
        _knn_kernel,
        grid=(B,),
        in_specs=[_bs((N, C_out)), _bs((N, 1)), _bs((1, N))],
        out_specs=[_bs((N, N)), _bs((N, _K)), _bs((1, 1))],
        out_shape=[jax.ShapeDtypeStruct((B, N, N), f32),
                   jax.ShapeDtypeStruct((B, N, _K), f32),
                   jax.ShapeDtypeStruct((B, 1, 1), f32)],
    )(xn, x2c, x2r)

    density = jnp.exp(-jnp.mean(d5 ** 2, -1))
    density = density + jax.random.uniform(jax.random.key(1), density.shape,
                                           f32) * 1e-6
    dc = density.reshape(B, N, 1)
    dr = density.reshape(B, 1, N)

    _chunk = pl.BlockSpec((1, 128, 1), lambda b, j: (b, j, 0))
    _chunkN = pl.BlockSpec((1, 128, N), lambda b, j: (b, j, 0))
    _row = pl.BlockSpec((1, 1, N), lambda b, j: (b, 0, 0))
    _one = pl.BlockSpec((1, 1, 1), lambda b, j: (b, 0, 0))

    score = pl.pallas_call(
        _score_kernel,
        grid=(B, N // 128),
        in_specs=[_chunkN, _chunk, _row, _one],
        out_specs=_chunk,
        out_shape=jax.ShapeDtypeStruct((B, N, 1), f32),
    )(dist, dc, dr, dmax)

    rank = pl.pallas_call(
        _rank_kernel,
        grid=(B, N // 128),
        in_specs=[_chunk, _row],
        out_specs=_chunk,
        out_shape=jax.ShapeDtypeStruct((B, N, 1), f32),
    )(score, score.reshape(B, 1, N))

    idxf = pl.pallas_call(
        _assign_kernel,
        grid=(B, N // 128),
        in_specs=[_chunkN, _chunk, _row],
        out_specs=_chunk,
        out_shape=jax.ShapeDtypeStruct((B, N, 1), f32),
    )(dist, rank, rank.reshape(B, 1, N))

    x_down = pl.pallas_call(
        _merge_kernel,
        grid=(B,),
        in_specs=[_bs((N, C_out)), _bs((N, 1)), _bs((N, 1))],
        out_specs=_bs((_CL, C_out)),
        out_shape=jax.ShapeDtypeStruct((B, _CL, C_out), f32),
    )(xn, idxf, ts)

    xm_img = xm.reshape(B, _H, _W, C_out)
    xmp = jnp.pad(xm_img, ((0, 0), (2, 2), (2, 2), (0, 0)))
    pp = [xmp[:, py::2, px::2, :] for py in range(2) for px in range(2)]
    dw_r = dw_w[:, 0, :, :].transpose(1, 2, 0).reshape(25, C_out)
    pw_t = pw_w[:, :, 0, 0].T
    x3 = fm

    x_kv = pl.pallas_call(
        _ms_kernel,
        grid=(B,),
        in_specs=[_bs((18, 18, C_out))] * 4 + [
            _bw((25, C_out)), _bw((1, C_out)), _bw((C_out, C_out)),
            _bw((1, C_out)), _bs((256, C_out)), _bw((1, C_out)),
            _bw((1, C_out))],
        out_specs=_bs((256, C_out)),
        out_shape=jax.ShapeDtypeStruct((B, 256, C_out), f32),
    )(pp[0], pp[1], pp[2], pp[3], dw_r, dw_b.reshape(1, C_out), pw_t,
      pw_b.reshape(1, C_out), x3, norm1_g.reshape(1, C_out),
      norm1_b.reshape(1, C_out))

    return x_down, x_kv, ts
